# baseline (speedup 1.0000x reference)
import jax
import jax.numpy as jnp
from jax import lax
from jax.experimental import pallas as pl
from jax.experimental.pallas import tpu as pltpu
from jax.experimental.pallas import tpu_sc as plsc

_N = 1048576
_C = 89
_NC = 2
_NS = 16
_NW = _NC * _NS
_EPW = _N // _NW
_EG = _EPW // 16
_MAP_PAD = 96
_NT = _MAP_PAD // 16

_BLK = 2048
_GRID = _N // _BLK


def _sc_gather_body(a_hbm, map_hbm, idx_hbm, map_v, a_v, idx_v):
    wid = lax.axis_index("s") * _NC + lax.axis_index("c")
    pltpu.sync_copy(map_hbm, map_v)
    base = wid * _EPW
    pltpu.sync_copy(a_hbm.at[pl.ds(base, _EPW)], a_v)

    tabs = [jnp.maximum(map_v[pl.ds(16 * k, 16)], 0) for k in range(_NT)]

    def step(g, carry):
        av = a_v[pl.ds(g * 16, 16)]
        av = jnp.clip(av, 0, _C)
        sub = av & 15
        hi = av >> 4
        idx = tabs[0].at[sub].get(mode="promise_in_bounds")
        for k in range(1, _NT):
            t = tabs[k].at[sub].get(mode="promise_in_bounds")
            idx = jnp.where(hi == k, t, idx)
        idx_v[pl.ds(g * 16, 16)] = idx
        return carry

    lax.fori_loop(0, _EG, step, 0)
    pltpu.sync_copy(idx_v, idx_hbm.at[pl.ds(base, _EPW)])


def _tc_onehot_body(idx_ref, out_ref):
    idx = idx_ref[0, 0, :]
    col = lax.broadcasted_iota(jnp.int32, (_BLK, _C), 1)
    out_ref[...] = (col == idx.reshape(_BLK, 1)).astype(jnp.float32)


def kernel(atomic_numbers, index_map):
    a = atomic_numbers.astype(jnp.int32)
    m = jnp.pad(index_map.astype(jnp.int32), (0, _MAP_PAD - index_map.shape[0]))

    mesh = plsc.VectorSubcoreMesh(core_axis_name="c", subcore_axis_name="s")
    idx = pl.kernel(
        _sc_gather_body,
        out_type=jax.ShapeDtypeStruct((_N,), jnp.int32),
        mesh=mesh,
        compiler_params=pltpu.CompilerParams(needs_layout_passes=False),
        scratch_types=[
            pltpu.VMEM((_MAP_PAD,), jnp.int32),
            pltpu.VMEM((_EPW,), jnp.int32),
            pltpu.VMEM((_EPW,), jnp.int32),
        ],
    )(a, m)

    return pl.pallas_call(
        _tc_onehot_body,
        out_shape=jax.ShapeDtypeStruct((_N, _C), jnp.float32),
        grid=(_GRID,),
        in_specs=[pl.BlockSpec((1, 1, _BLK), lambda g: (g, 0, 0))],
        out_specs=pl.BlockSpec((_BLK, _C), lambda g: (g, 0)),
        compiler_params=pltpu.CompilerParams(
            dimension_semantics=("parallel",),
        ),
    )(idx.reshape(_GRID, 1, _BLK))


# speedup vs baseline: 1.3256x; 1.3256x over previous
import jax
import jax.numpy as jnp
from jax import lax
from jax.experimental import pallas as pl
from jax.experimental.pallas import tpu as pltpu
from jax.experimental.pallas import tpu_sc as plsc

_N = 1048576
_C = 89
_NC = 2
_NS = 16
_NW = _NC * _NS
_EPW = _N // _NW
_EG = _EPW // 16
_MAP_PAD = 96
_NT = _MAP_PAD // 16

_BLK = 16384
_GRID = _N // _BLK


def _sc_gather_body(a_hbm, map_hbm, idx_hbm, map_v, a_v, idx_v):
    wid = lax.axis_index("s") * _NC + lax.axis_index("c")
    pltpu.sync_copy(map_hbm, map_v)
    base = wid * _EPW
    pltpu.sync_copy(a_hbm.at[pl.ds(base, _EPW)], a_v)

    tabs = [jnp.maximum(map_v[pl.ds(16 * k, 16)], 0) for k in range(_NT)]

    def step(g, carry):
        av = a_v[pl.ds(g * 16, 16)]
        av = jnp.clip(av, 0, _C)
        sub = av & 15
        hi = av >> 4
        idx = tabs[0].at[sub].get(mode="promise_in_bounds")
        for k in range(1, _NT):
            t = tabs[k].at[sub].get(mode="promise_in_bounds")
            idx = jnp.where(hi == k, t, idx)
        idx_v[pl.ds(g * 16, 16)] = idx
        return carry

    lax.fori_loop(0, _EG, step, 0)
    pltpu.sync_copy(idx_v, idx_hbm.at[pl.ds(base, _EPW)])


def _tc_onehot_body(idx_ref, out_ref):
    idx = idx_ref[0, 0, :]
    col = lax.broadcasted_iota(jnp.int32, (_BLK, _C), 1)
    out_ref[...] = (col == idx.reshape(_BLK, 1)).astype(jnp.float32)


def kernel(atomic_numbers, index_map):
    a = atomic_numbers.astype(jnp.int32)
    m = jnp.pad(index_map.astype(jnp.int32), (0, _MAP_PAD - index_map.shape[0]))

    mesh = plsc.VectorSubcoreMesh(core_axis_name="c", subcore_axis_name="s")
    idx = pl.kernel(
        _sc_gather_body,
        out_type=jax.ShapeDtypeStruct((_N,), jnp.int32),
        mesh=mesh,
        compiler_params=pltpu.CompilerParams(needs_layout_passes=False),
        scratch_types=[
            pltpu.VMEM((_MAP_PAD,), jnp.int32),
            pltpu.VMEM((_EPW,), jnp.int32),
            pltpu.VMEM((_EPW,), jnp.int32),
        ],
    )(a, m)

    return pl.pallas_call(
        _tc_onehot_body,
        out_shape=jax.ShapeDtypeStruct((_N, _C), jnp.float32),
        grid=(_GRID,),
        in_specs=[pl.BlockSpec((1, 1, _BLK), lambda g: (g, 0, 0))],
        out_specs=pl.BlockSpec((_BLK, _C), lambda g: (g, 0)),
        compiler_params=pltpu.CompilerParams(
            dimension_semantics=("parallel",),
        ),
    )(idx.reshape(_GRID, 1, _BLK))


# speedup vs baseline: 1.3357x; 1.0076x over previous
import jax
import jax.numpy as jnp
from jax import lax
from jax.experimental import pallas as pl
from jax.experimental.pallas import tpu as pltpu
from jax.experimental.pallas import tpu_sc as plsc

_N = 1048576
_C = 89
_NC = 2
_NS = 16
_NW = _NC * _NS
_EPW = _N // _NW
_EG = _EPW // 16
_MAP_PAD = 96
_NT = _MAP_PAD // 16

_BLK = 32768
_GRID = _N // _BLK


def _sc_gather_body(a_hbm, map_hbm, idx_hbm, map_v, a_v, idx_v):
    wid = lax.axis_index("s") * _NC + lax.axis_index("c")
    pltpu.sync_copy(map_hbm, map_v)
    base = wid * _EPW
    pltpu.sync_copy(a_hbm.at[pl.ds(base, _EPW)], a_v)

    tabs = [jnp.maximum(map_v[pl.ds(16 * k, 16)], 0) for k in range(_NT)]

    def step(g, carry):
        av = a_v[pl.ds(g * 16, 16)]
        av = jnp.clip(av, 0, _C)
        sub = av & 15
        hi = av >> 4
        idx = tabs[0].at[sub].get(mode="promise_in_bounds")
        for k in range(1, _NT):
            t = tabs[k].at[sub].get(mode="promise_in_bounds")
            idx = jnp.where(hi == k, t, idx)
        idx_v[pl.ds(g * 16, 16)] = idx
        return carry

    lax.fori_loop(0, _EG, step, 0)
    pltpu.sync_copy(idx_v, idx_hbm.at[pl.ds(base, _EPW)])


def _tc_onehot_body(idx_ref, out_ref):
    idx = idx_ref[0, 0, :]
    col = lax.broadcasted_iota(jnp.int32, (_BLK, _C), 1)
    out_ref[...] = (col == idx.reshape(_BLK, 1)).astype(jnp.float32)


def kernel(atomic_numbers, index_map):
    a = atomic_numbers.astype(jnp.int32)
    m = jnp.pad(index_map.astype(jnp.int32), (0, _MAP_PAD - index_map.shape[0]))

    mesh = plsc.VectorSubcoreMesh(core_axis_name="c", subcore_axis_name="s")
    idx = pl.kernel(
        _sc_gather_body,
        out_type=jax.ShapeDtypeStruct((_N,), jnp.int32),
        mesh=mesh,
        compiler_params=pltpu.CompilerParams(needs_layout_passes=False),
        scratch_types=[
            pltpu.VMEM((_MAP_PAD,), jnp.int32),
            pltpu.VMEM((_EPW,), jnp.int32),
            pltpu.VMEM((_EPW,), jnp.int32),
        ],
    )(a, m)

    return pl.pallas_call(
        _tc_onehot_body,
        out_shape=jax.ShapeDtypeStruct((_N, _C), jnp.float32),
        grid=(_GRID,),
        in_specs=[pl.BlockSpec((1, 1, _BLK), lambda g: (g, 0, 0))],
        out_specs=pl.BlockSpec((_BLK, _C), lambda g: (g, 0)),
        compiler_params=pltpu.CompilerParams(
            dimension_semantics=("parallel",),
        ),
    )(idx.reshape(_GRID, 1, _BLK))
